# Initial kernel scaffold; baseline (speedup 1.0000x reference)
import jax
import jax.numpy as jnp
from jax.experimental import pallas as pl


def kernel(x, edge_index, Wq1, bq1, Wq2, bq2, Wp1, bp1, Wp2, bp2, t_final):
    raise NotImplementedError("write your pallas kernel here")


# speedup vs baseline: 54.0719x; 54.0719x over previous
import functools

import jax
import jax.numpy as jnp
from jax import lax
from jax.experimental import pallas as pl
from jax.experimental.pallas import tpu as pltpu
from jax.experimental.pallas import tpu_sc as plsc

N = 50000
E = 800000
DT = 1.0
N_PAD = 50176
E_PAD = 819200
IDX_ROWS = E_PAD // 128
PER_W = E_PAD // 32
CHUNKS = PER_W // 1024
ROWS_PER_TILE = N_PAD // 16


def _sc_mesh():
    return plsc.VectorSubcoreMesh(
        core_axis_name="c", subcore_axis_name="s", num_cores=2, num_subcores=16
    )


def _gather(table, i0p, i1p):
    @functools.partial(
        pl.kernel,
        out_type=[jax.ShapeDtypeStruct((E_PAD, 32), jnp.float32)] * 2,
        mesh=_sc_mesh(),
        compiler_params=pltpu.CompilerParams(use_tc_tiling_on_sc=False),
        scratch_types=[
            pltpu.VMEM((8, 128), jnp.int32),
            pltpu.VMEM((8, 128), jnp.int32),
            pltpu.VMEM((1024, 32), jnp.float32),
            pltpu.VMEM((1024, 32), jnp.float32),
            pltpu.SemaphoreType.DMA,
            pltpu.SemaphoreType.DMA,
        ],
    )
    def k(table_h, i0_h, i1_h, fs_h, fd_h, ia_v, ib_v, ra_v, rb_v, sa, sb):
        wid = lax.axis_index("s") * 2 + lax.axis_index("c")

        @pl.loop(0, CHUNKS)
        def _chunk(kk):
            cb = wid * PER_W + kk * 1024
            irow = wid * (PER_W // 128) + kk * 8
            pltpu.sync_copy(i0_h.at[pl.ds(irow, 8)], ia_v)
            pltpu.sync_copy(i1_h.at[pl.ds(irow, 8)], ib_v)
            cps = [
                pltpu.async_copy(table_h.at[ia_v.at[j]], ra_v.at[pl.ds(j * 128, 128)], sa)
                for j in range(8)
            ] + [
                pltpu.async_copy(table_h.at[ib_v.at[j]], rb_v.at[pl.ds(j * 128, 128)], sb)
                for j in range(8)
            ]
            for c in cps:
                c.wait()
            pltpu.sync_copy(ra_v, fs_h.at[pl.ds(cb, 1024)])
            pltpu.sync_copy(rb_v, fd_h.at[pl.ds(cb, 1024)])

    return k(table, i0p, i1p)


def _scatter(gs, gd, i0p, i1p, zrows):
    @functools.partial(
        pl.kernel,
        out_type=jax.ShapeDtypeStruct((2, N_PAD, 8), jnp.float32),
        mesh=_sc_mesh(),
        compiler_params=pltpu.CompilerParams(use_tc_tiling_on_sc=False),
        scratch_types=[
            pltpu.VMEM_SHARED((N_PAD, 8), jnp.float32),
            pltpu.VMEM((8, 128), jnp.int32),
            pltpu.VMEM((8, 128), jnp.int32),
            pltpu.VMEM((1024, 8), jnp.float32),
            pltpu.VMEM((1024, 8), jnp.float32),
        ],
    )
    def k(gs_h, gd_h, i0_h, i1_h, z_h, out_h, acc_sh, ia_v, ib_v, sa_v, sb_v):
        cid = lax.axis_index("c")
        sid = lax.axis_index("s")
        wid = sid * 2 + cid
        pltpu.sync_copy(z_h, acc_sh.at[pl.ds(sid * ROWS_PER_TILE, ROWS_PER_TILE)])
        plsc.subcore_barrier()

        @pl.loop(0, CHUNKS)
        def _chunk(kk):
            cb = wid * PER_W + kk * 1024
            irow = wid * (PER_W // 128) + kk * 8
            pltpu.sync_copy(i0_h.at[pl.ds(irow, 8)], ia_v)
            pltpu.sync_copy(i1_h.at[pl.ds(irow, 8)], ib_v)
            pltpu.sync_copy(gs_h.at[pl.ds(cb, 1024)], sa_v)
            pltpu.sync_copy(gd_h.at[pl.ds(cb, 1024)], sb_v)
            for j in range(8):
                pltpu.sync_copy(sa_v.at[pl.ds(j * 128, 128)], acc_sh.at[ia_v.at[j]], add=True)
            for j in range(8):
                pltpu.sync_copy(sb_v.at[pl.ds(j * 128, 128)], acc_sh.at[ib_v.at[j]], add=True)

        plsc.subcore_barrier()
        pltpu.sync_copy(
            acc_sh.at[pl.ds(sid * ROWS_PER_TILE, ROWS_PER_TILE)],
            out_h.at[cid, pl.ds(sid * ROWS_PER_TILE, ROWS_PER_TILE)],
        )

    return k(gs, gd, i0p, i1p, zrows)


def _mlp_body(fs_r, fd_r, ws_r, wd_r, bw_r, w2_r, gi_r, gj_r, gs_o, gd_o):
    z = jnp.dot(fs_r[...], ws_r[...], preferred_element_type=jnp.float32)
    z = z + jnp.dot(fd_r[...], wd_r[...], preferred_element_type=jnp.float32)
    z = z + bw_r[...]
    h = jnp.tanh(z)
    g = (1.0 - h * h) * w2_r[...]
    gs_o[...] = jnp.dot(g, gi_r[...], preferred_element_type=jnp.float32)
    gd_o[...] = jnp.dot(g, gj_r[...], preferred_element_type=jnp.float32)


def _mlp(fs, fd, ws, wd, bw, w2, gi, gj):
    r = 4096
    grid = (E_PAD // r,)
    full = lambda shape: pl.BlockSpec(shape, lambda i: (0, 0))
    return pl.pallas_call(
        _mlp_body,
        grid=grid,
        in_specs=[
            pl.BlockSpec((r, 32), lambda i: (i, 0)),
            pl.BlockSpec((r, 32), lambda i: (i, 0)),
            full((32, 64)),
            full((32, 64)),
            full((1, 64)),
            full((1, 64)),
            full((64, 8)),
            full((64, 8)),
        ],
        out_specs=[
            pl.BlockSpec((r, 8), lambda i: (i, 0)),
            pl.BlockSpec((r, 8), lambda i: (i, 0)),
        ],
        out_shape=[jax.ShapeDtypeStruct((E_PAD, 8), jnp.float32)] * 2,
    )(fs, fd, ws, wd, bw, w2, gi, gj)


def _make_mats(W1, b1, W2):
    ws = jnp.zeros((32, 64), jnp.float32)
    ws = ws.at[0:4, 0:32].set(W1[0:4])
    ws = ws.at[4:12, 0:32].set(W1[8:16])
    ws = ws.at[16:20, 32:64].set(W1[0:4])
    ws = ws.at[20:28, 32:64].set(W1[8:16])
    wd = jnp.zeros((32, 64), jnp.float32)
    wd = wd.at[0:4, 0:32].set(W1[4:8])
    wd = wd.at[4:12, 0:32].set(W1[16:24])
    wd = wd.at[16:20, 32:64].set(W1[4:8])
    wd = wd.at[20:28, 32:64].set(W1[16:24])
    bw = jnp.concatenate([b1, b1]).reshape(1, 64)
    w2 = jnp.concatenate([W2[:, 0], W2[:, 0]]).reshape(1, 64)
    gi = jnp.zeros((64, 8), jnp.float32)
    gi = gi.at[0:32, 0:4].set(W1[0:4].T)
    gi = gi.at[32:64, 4:8].set(W1[0:4].T)
    gj = jnp.zeros((64, 8), jnp.float32)
    gj = gj.at[0:32, 0:4].set(W1[4:8].T)
    gj = gj.at[32:64, 4:8].set(W1[4:8].T)
    return ws, wd, bw, w2, gi, gj


def kernel(x, edge_index, Wq1, bq1, Wq2, bq2, Wp1, bp1, Wp2, bp2, t_final):
    q = x[..., 0:4]
    p = x[..., 4:8]
    xi = x[..., 8:16]

    pad = jnp.full((E_PAD - E,), N, jnp.int32)
    i0p = jnp.concatenate([edge_index[0], pad]).reshape(IDX_ROWS, 128)
    i1p = jnp.concatenate([edge_index[1], pad]).reshape(IDX_ROWS, 128)
    zrows = jnp.zeros((ROWS_PER_TILE, 8), jnp.float32)

    mats_q = _make_mats(Wq1, bq1, Wq2)
    mats_p = _make_mats(Wp1, bp1, Wp2)

    xi_cols = jnp.zeros((N_PAD, 32), jnp.float32)
    xi_cols = xi_cols.at[:N, 4:12].set(xi[0])
    xi_cols = xi_cols.at[:N, 20:28].set(xi[1])

    def grad_eval(var, mats):
        table = xi_cols.at[:N, 0:4].set(var[0]).at[:N, 16:20].set(var[1])
        fs, fd = _gather(table, i0p, i1p)
        gs, gd = _mlp(fs, fd, *mats)
        parts = _scatter(gs, gd, i0p, i1p, zrows)
        acc = parts[0] + parts[1]
        return jnp.stack([acc[:N, 0:4], acc[:N, 4:8]])

    for _ in range(2):
        p = p - DT * grad_eval(q, mats_q)
        q = q + DT * grad_eval(p, mats_p)

    out = jnp.concatenate([q, p, xi], axis=-1)
    return out + jnp.asarray(t_final * 0, dtype=out.dtype)


# speedup vs baseline: 65.2260x; 1.2063x over previous
import functools

import jax
import jax.numpy as jnp
from jax import lax
from jax.experimental import pallas as pl
from jax.experimental.pallas import tpu as pltpu
from jax.experimental.pallas import tpu_sc as plsc

N = 50000
E = 800000
DT = 1.0
N_PAD = 50176
E_PAD = 819200
IDX_ROWS = E_PAD // 128
PER_W = E_PAD // 32
CHUNKS = PER_W // 1024
IROWS_W = PER_W // 128
ROWS_PER_TILE = N_PAD // 16


def _sc_mesh():
    return plsc.VectorSubcoreMesh(
        core_axis_name="c", subcore_axis_name="s", num_cores=2, num_subcores=16
    )


def _gather(table, i0p, i1p):
    @functools.partial(
        pl.kernel,
        out_type=[jax.ShapeDtypeStruct((E_PAD, 32), jnp.float32)] * 2,
        mesh=_sc_mesh(),
        compiler_params=pltpu.CompilerParams(use_tc_tiling_on_sc=False),
        scratch_types=[
            pltpu.VMEM((IROWS_W, 128), jnp.int32),
            pltpu.VMEM((IROWS_W, 128), jnp.int32),
            pltpu.VMEM((1024, 32), jnp.float32),
            pltpu.VMEM((1024, 32), jnp.float32),
            pltpu.SemaphoreType.DMA,
            pltpu.SemaphoreType.DMA,
        ],
    )
    def k(table_h, i0_h, i1_h, fs_h, fd_h, ia_v, ib_v, ra_v, rb_v, sa, sb):
        wid = lax.axis_index("s") * 2 + lax.axis_index("c")
        pltpu.sync_copy(i0_h.at[pl.ds(wid * IROWS_W, IROWS_W)], ia_v)
        pltpu.sync_copy(i1_h.at[pl.ds(wid * IROWS_W, IROWS_W)], ib_v)

        @pl.loop(0, CHUNKS)
        def _chunk(kk):
            cb = wid * PER_W + kk * 1024
            cps = [
                pltpu.async_copy(table_h.at[ia_v.at[kk * 8 + j]], ra_v.at[pl.ds(j * 128, 128)], sa)
                for j in range(8)
            ] + [
                pltpu.async_copy(table_h.at[ib_v.at[kk * 8 + j]], rb_v.at[pl.ds(j * 128, 128)], sb)
                for j in range(8)
            ]
            for c in cps:
                c.wait()
            pltpu.sync_copy(ra_v, fs_h.at[pl.ds(cb, 1024)])
            pltpu.sync_copy(rb_v, fd_h.at[pl.ds(cb, 1024)])

    return k(table, i0p, i1p)


def _scatter(gs, gd, i0p, i1p, zrows):
    @functools.partial(
        pl.kernel,
        out_type=jax.ShapeDtypeStruct((2, N_PAD, 8), jnp.float32),
        mesh=_sc_mesh(),
        compiler_params=pltpu.CompilerParams(use_tc_tiling_on_sc=False),
        scratch_types=[
            pltpu.VMEM_SHARED((N_PAD, 8), jnp.float32),
            pltpu.VMEM((IROWS_W, 128), jnp.int32),
            pltpu.VMEM((IROWS_W, 128), jnp.int32),
            pltpu.VMEM((1024, 8), jnp.float32),
            pltpu.VMEM((1024, 8), jnp.float32),
            pltpu.VMEM((1024, 8), jnp.float32),
            pltpu.VMEM((1024, 8), jnp.float32),
            pltpu.SemaphoreType.DMA,
            pltpu.SemaphoreType.DMA,
        ],
    )
    def k(gs_h, gd_h, i0_h, i1_h, z_h, out_h,
          acc_sh, ia_v, ib_v, sa0, sb0, sa1, sb1, s_ld, s_add):
        cid = lax.axis_index("c")
        sid = lax.axis_index("s")
        wid = sid * 2 + cid
        pltpu.sync_copy(z_h, acc_sh.at[pl.ds(sid * ROWS_PER_TILE, ROWS_PER_TILE)])
        pltpu.sync_copy(i0_h.at[pl.ds(wid * IROWS_W, IROWS_W)], ia_v)
        pltpu.sync_copy(i1_h.at[pl.ds(wid * IROWS_W, IROWS_W)], ib_v)
        plsc.subcore_barrier()

        def load(kk, sa, sb):
            cb = wid * PER_W + kk * 1024
            return [
                pltpu.async_copy(gs_h.at[pl.ds(cb, 1024)], sa, s_ld),
                pltpu.async_copy(gd_h.at[pl.ds(cb, 1024)], sb, s_ld),
            ]

        def adds(kk, sa, sb):
            cps = [
                pltpu.async_copy(sa.at[pl.ds(j * 128, 128)], acc_sh.at[ia_v.at[kk * 8 + j]],
                                 s_add, add=True)
                for j in range(8)
            ] + [
                pltpu.async_copy(sb.at[pl.ds(j * 128, 128)], acc_sh.at[ib_v.at[kk * 8 + j]],
                                 s_add, add=True)
                for j in range(8)
            ]
            for c in cps:
                c.wait()

        for c in load(0, sa0, sb0):
            c.wait()

        @pl.loop(0, (CHUNKS - 1) // 2)
        def _pair(t):
            ka = 2 * t + 1
            l1 = load(ka, sa1, sb1)
            adds(2 * t, sa0, sb0)
            for c in l1:
                c.wait()
            l0 = load(ka + 1, sa0, sb0)
            adds(ka, sa1, sb1)
            for c in l0:
                c.wait()

        adds(CHUNKS - 1, sa0, sb0)

        plsc.subcore_barrier()
        pltpu.sync_copy(
            acc_sh.at[pl.ds(sid * ROWS_PER_TILE, ROWS_PER_TILE)],
            out_h.at[cid, pl.ds(sid * ROWS_PER_TILE, ROWS_PER_TILE)],
        )

    return k(gs, gd, i0p, i1p, zrows)


def _mlp_body(fs_r, fd_r, ws_r, wd_r, bw_r, w2_r, gi_r, gj_r, gs_o, gd_o):
    z = jnp.dot(fs_r[...], ws_r[...], preferred_element_type=jnp.float32)
    z = z + jnp.dot(fd_r[...], wd_r[...], preferred_element_type=jnp.float32)
    z = z + bw_r[...]
    h = jnp.tanh(z)
    g = (1.0 - h * h) * w2_r[...]
    gs_o[...] = jnp.dot(g, gi_r[...], preferred_element_type=jnp.float32)
    gd_o[...] = jnp.dot(g, gj_r[...], preferred_element_type=jnp.float32)


def _mlp(fs, fd, ws, wd, bw, w2, gi, gj):
    r = 4096
    full = lambda shape: pl.BlockSpec(shape, lambda i: (0, 0))
    return pl.pallas_call(
        _mlp_body,
        grid=(E_PAD // r,),
        in_specs=[
            pl.BlockSpec((r, 32), lambda i: (i, 0)),
            pl.BlockSpec((r, 32), lambda i: (i, 0)),
            full((32, 64)),
            full((32, 64)),
            full((1, 64)),
            full((1, 64)),
            full((64, 8)),
            full((64, 8)),
        ],
        out_specs=[
            pl.BlockSpec((r, 8), lambda i: (i, 0)),
            pl.BlockSpec((r, 8), lambda i: (i, 0)),
        ],
        out_shape=[jax.ShapeDtypeStruct((E_PAD, 8), jnp.float32)] * 2,
    )(fs, fd, ws, wd, bw, w2, gi, gj)


def _update(table, parts, cols, sign):
    r = 3136
    c0, c1 = cols

    def body(t_r, pa_r, o_r):
        acc = pa_r[0] + pa_r[1]
        t = t_r[...]
        d0 = sign * DT * acc[:, 0:4]
        d1 = sign * DT * acc[:, 4:8]
        pieces = []
        if c0 > 0:
            pieces.append(t[:, 0:c0])
        pieces.append(t[:, c0:c0 + 4] + d0)
        pieces.append(t[:, c0 + 4:c1])
        pieces.append(t[:, c1:c1 + 4] + d1)
        if c1 + 4 < 32:
            pieces.append(t[:, c1 + 4:32])
        o_r[...] = jnp.concatenate(pieces, axis=1)

    return pl.pallas_call(
        body,
        grid=(N_PAD // r,),
        in_specs=[
            pl.BlockSpec((r, 32), lambda i: (i, 0)),
            pl.BlockSpec((2, r, 8), lambda i: (0, i, 0)),
        ],
        out_specs=pl.BlockSpec((r, 32), lambda i: (i, 0)),
        out_shape=jax.ShapeDtypeStruct((N_PAD, 32), jnp.float32),
    )(table, parts)


def _make_mats(W1, b1, W2, vo):
    ws = jnp.zeros((32, 64), jnp.float32)
    wd = jnp.zeros((32, 64), jnp.float32)
    for bo, co in ((0, 0), (16, 32)):
        ws = ws.at[bo + vo:bo + vo + 4, co:co + 32].set(W1[0:4])
        ws = ws.at[bo + 8:bo + 16, co:co + 32].set(W1[8:16])
        wd = wd.at[bo + vo:bo + vo + 4, co:co + 32].set(W1[4:8])
        wd = wd.at[bo + 8:bo + 16, co:co + 32].set(W1[16:24])
    bw = jnp.concatenate([b1, b1]).reshape(1, 64)
    w2 = jnp.concatenate([W2[:, 0], W2[:, 0]]).reshape(1, 64)
    gi = jnp.zeros((64, 8), jnp.float32)
    gi = gi.at[0:32, 0:4].set(W1[0:4].T)
    gi = gi.at[32:64, 4:8].set(W1[0:4].T)
    gj = jnp.zeros((64, 8), jnp.float32)
    gj = gj.at[0:32, 0:4].set(W1[4:8].T)
    gj = gj.at[32:64, 4:8].set(W1[4:8].T)
    return ws, wd, bw, w2, gi, gj


def kernel(x, edge_index, Wq1, bq1, Wq2, bq2, Wp1, bp1, Wp2, bp2, t_final):
    q = x[..., 0:4]
    p = x[..., 4:8]
    xi = x[..., 8:16]

    pad = jnp.full((E_PAD - E,), N, jnp.int32)
    i0p = jnp.concatenate([edge_index[0], pad]).reshape(IDX_ROWS, 128)
    i1p = jnp.concatenate([edge_index[1], pad]).reshape(IDX_ROWS, 128)
    zrows = jnp.zeros((ROWS_PER_TILE, 8), jnp.float32)

    mats_q = _make_mats(Wq1, bq1, Wq2, 0)
    mats_p = _make_mats(Wp1, bp1, Wp2, 4)

    body = jnp.concatenate([q[0], p[0], xi[0], q[1], p[1], xi[1]], axis=1)
    table = jnp.concatenate([body, jnp.zeros((N_PAD - N, 32), jnp.float32)], axis=0)

    def grad_parts(table, mats):
        fs, fd = _gather(table, i0p, i1p)
        gs, gd = _mlp(fs, fd, *mats)
        return _scatter(gs, gd, i0p, i1p, zrows)

    for _ in range(2):
        table = _update(table, grad_parts(table, mats_q), (4, 20), -1.0)
        table = _update(table, grad_parts(table, mats_p), (0, 16), 1.0)

    out = jnp.stack([table[:N, 0:16], table[:N, 16:32]])
    return out + jnp.asarray(t_final * 0, dtype=out.dtype)


# speedup vs baseline: 136.7223x; 2.0961x over previous
import functools

import jax
import jax.numpy as jnp
from jax import lax
from jax.experimental import pallas as pl
from jax.experimental.pallas import tpu as pltpu
from jax.experimental.pallas import tpu_sc as plsc

N = 50000
E = 800000
DT = 1.0
N_PAD = 50176
E_PAD = 819200
IDX_ROWS = E_PAD // 128
PER_W = E_PAD // 32
CHUNKS = PER_W // 1024
IROWS_W = PER_W // 128
ROWS_PER_TILE = N_PAD // 16


def _sc_mesh():
    return plsc.VectorSubcoreMesh(
        core_axis_name="c", subcore_axis_name="s", num_cores=2, num_subcores=16
    )


def _gather(table, i0p, i1p):
    @functools.partial(
        pl.kernel,
        out_type=[jax.ShapeDtypeStruct((E_PAD, 32), jnp.float32)] * 2,
        mesh=_sc_mesh(),
        compiler_params=pltpu.CompilerParams(use_tc_tiling_on_sc=False),
        scratch_types=[
            pltpu.VMEM((IROWS_W, 128), jnp.int32),
            pltpu.VMEM((IROWS_W, 128), jnp.int32),
            pltpu.VMEM((1024, 32), jnp.float32),
            pltpu.VMEM((1024, 32), jnp.float32),
            pltpu.SemaphoreType.DMA,
            pltpu.SemaphoreType.DMA,
        ],
    )
    def k(table_h, i0_h, i1_h, fs_h, fd_h, ia_v, ib_v, ra_v, rb_v, sa, sb):
        wid = lax.axis_index("s") * 2 + lax.axis_index("c")
        pltpu.sync_copy(i0_h.at[pl.ds(wid * IROWS_W, IROWS_W)], ia_v)
        pltpu.sync_copy(i1_h.at[pl.ds(wid * IROWS_W, IROWS_W)], ib_v)

        @pl.loop(0, CHUNKS)
        def _chunk(kk):
            cb = wid * PER_W + kk * 1024
            cps = [
                pltpu.async_copy(table_h.at[ia_v.at[kk * 8 + j]], ra_v.at[pl.ds(j * 128, 128)], sa)
                for j in range(8)
            ] + [
                pltpu.async_copy(table_h.at[ib_v.at[kk * 8 + j]], rb_v.at[pl.ds(j * 128, 128)], sb)
                for j in range(8)
            ]
            for c in cps:
                c.wait()
            pltpu.sync_copy(ra_v, fs_h.at[pl.ds(cb, 1024)])
            pltpu.sync_copy(rb_v, fd_h.at[pl.ds(cb, 1024)])

    return k(table, i0p, i1p)


def _scatter(gs, gd, i0p, i1p, zrows):
    @functools.partial(
        pl.kernel,
        out_type=jax.ShapeDtypeStruct((2, N_PAD, 8), jnp.float32),
        mesh=_sc_mesh(),
        compiler_params=pltpu.CompilerParams(use_tc_tiling_on_sc=False),
        scratch_types=[
            pltpu.VMEM_SHARED((N_PAD, 8), jnp.float32),
            pltpu.VMEM((IROWS_W, 128), jnp.int32),
            pltpu.VMEM((IROWS_W, 128), jnp.int32),
            pltpu.VMEM((1024, 8), jnp.float32),
            pltpu.VMEM((1024, 8), jnp.float32),
            pltpu.VMEM((1024, 8), jnp.float32),
            pltpu.VMEM((1024, 8), jnp.float32),
            pltpu.SemaphoreType.DMA,
            pltpu.SemaphoreType.DMA,
        ],
    )
    def k(gs_h, gd_h, i0_h, i1_h, z_h, out_h,
          acc_sh, ia_v, ib_v, sa0, sb0, sa1, sb1, s_ld, s_add):
        cid = lax.axis_index("c")
        sid = lax.axis_index("s")
        wid = sid * 2 + cid
        pltpu.sync_copy(z_h, acc_sh.at[pl.ds(sid * ROWS_PER_TILE, ROWS_PER_TILE)])
        pltpu.sync_copy(i0_h.at[pl.ds(wid * IROWS_W, IROWS_W)], ia_v)
        pltpu.sync_copy(i1_h.at[pl.ds(wid * IROWS_W, IROWS_W)], ib_v)
        plsc.subcore_barrier()

        def load(kk, sa, sb):
            cb = wid * PER_W + kk * 1024
            return [
                pltpu.async_copy(gs_h.at[pl.ds(cb, 1024)], sa, s_ld),
                pltpu.async_copy(gd_h.at[pl.ds(cb, 1024)], sb, s_ld),
            ]

        def adds(kk, sa, sb):
            cps = [
                pltpu.async_copy(sa.at[pl.ds(j * 128, 128)], acc_sh.at[ia_v.at[kk * 8 + j]],
                                 s_add, add=True)
                for j in range(8)
            ] + [
                pltpu.async_copy(sb.at[pl.ds(j * 128, 128)], acc_sh.at[ib_v.at[kk * 8 + j]],
                                 s_add, add=True)
                for j in range(8)
            ]
            for c in cps:
                c.wait()

        for c in load(0, sa0, sb0):
            c.wait()

        @pl.loop(0, (CHUNKS - 1) // 2)
        def _pair(t):
            ka = 2 * t + 1
            l1 = load(ka, sa1, sb1)
            adds(2 * t, sa0, sb0)
            for c in l1:
                c.wait()
            l0 = load(ka + 1, sa0, sb0)
            adds(ka, sa1, sb1)
            for c in l0:
                c.wait()

        adds(CHUNKS - 1, sa0, sb0)

        plsc.subcore_barrier()
        pltpu.sync_copy(
            acc_sh.at[pl.ds(sid * ROWS_PER_TILE, ROWS_PER_TILE)],
            out_h.at[cid, pl.ds(sid * ROWS_PER_TILE, ROWS_PER_TILE)],
        )

    return k(gs, gd, i0p, i1p, zrows)


def _mlp_body(fs_r, fd_r, ws_r, wd_r, bw_r, w2_r, gi_r, gj_r, gs_o, gd_o):
    z = jnp.dot(fs_r[...], ws_r[...], preferred_element_type=jnp.float32)
    z = z + jnp.dot(fd_r[...], wd_r[...], preferred_element_type=jnp.float32)
    z = z + bw_r[...]
    h = jnp.tanh(z)
    g = (1.0 - h * h) * w2_r[...]
    gs_o[...] = jnp.dot(g, gi_r[...], preferred_element_type=jnp.float32)
    gd_o[...] = jnp.dot(g, gj_r[...], preferred_element_type=jnp.float32)


def _mlp(fs, fd, ws, wd, bw, w2, gi, gj):
    r = 2048
    rows = E_PAD // 4
    fs4 = fs.reshape(rows, 128)
    fd4 = fd.reshape(rows, 128)
    eye4 = jnp.eye(4, dtype=jnp.float32)
    ws4 = jnp.kron(eye4, ws)
    wd4 = jnp.kron(eye4, wd)
    bw4 = jnp.tile(bw, (1, 4))
    w24 = jnp.tile(w2, (1, 4))
    gi4 = jnp.kron(eye4, gi)
    gj4 = jnp.kron(eye4, gj)
    full = lambda shape: pl.BlockSpec(shape, lambda i: (0, 0))
    gs4, gd4 = pl.pallas_call(
        _mlp_body,
        grid=(rows // r,),
        in_specs=[
            pl.BlockSpec((r, 128), lambda i: (i, 0)),
            pl.BlockSpec((r, 128), lambda i: (i, 0)),
            full((128, 256)),
            full((128, 256)),
            full((1, 256)),
            full((1, 256)),
            full((256, 32)),
            full((256, 32)),
        ],
        out_specs=[
            pl.BlockSpec((r, 32), lambda i: (i, 0)),
            pl.BlockSpec((r, 32), lambda i: (i, 0)),
        ],
        out_shape=[jax.ShapeDtypeStruct((rows, 32), jnp.float32)] * 2,
    )(fs4, fd4, ws4, wd4, bw4, w24, gi4, gj4)
    return gs4.reshape(E_PAD, 8), gd4.reshape(E_PAD, 8)


def _update(table, parts, cols, sign):
    r = 3136
    c0, c1 = cols

    def body(t_r, pa_r, o_r):
        acc = pa_r[0] + pa_r[1]
        t = t_r[...]
        d0 = sign * DT * acc[:, 0:4]
        d1 = sign * DT * acc[:, 4:8]
        pieces = []
        if c0 > 0:
            pieces.append(t[:, 0:c0])
        pieces.append(t[:, c0:c0 + 4] + d0)
        pieces.append(t[:, c0 + 4:c1])
        pieces.append(t[:, c1:c1 + 4] + d1)
        if c1 + 4 < 32:
            pieces.append(t[:, c1 + 4:32])
        o_r[...] = jnp.concatenate(pieces, axis=1)

    return pl.pallas_call(
        body,
        grid=(N_PAD // r,),
        in_specs=[
            pl.BlockSpec((r, 32), lambda i: (i, 0)),
            pl.BlockSpec((2, r, 8), lambda i: (0, i, 0)),
        ],
        out_specs=pl.BlockSpec((r, 32), lambda i: (i, 0)),
        out_shape=jax.ShapeDtypeStruct((N_PAD, 32), jnp.float32),
    )(table, parts)


def _make_mats(W1, b1, W2, vo):
    ws = jnp.zeros((32, 64), jnp.float32)
    wd = jnp.zeros((32, 64), jnp.float32)
    for bo, co in ((0, 0), (16, 32)):
        ws = ws.at[bo + vo:bo + vo + 4, co:co + 32].set(W1[0:4])
        ws = ws.at[bo + 8:bo + 16, co:co + 32].set(W1[8:16])
        wd = wd.at[bo + vo:bo + vo + 4, co:co + 32].set(W1[4:8])
        wd = wd.at[bo + 8:bo + 16, co:co + 32].set(W1[16:24])
    bw = jnp.concatenate([b1, b1]).reshape(1, 64)
    w2 = jnp.concatenate([W2[:, 0], W2[:, 0]]).reshape(1, 64)
    gi = jnp.zeros((64, 8), jnp.float32)
    gi = gi.at[0:32, 0:4].set(W1[0:4].T)
    gi = gi.at[32:64, 4:8].set(W1[0:4].T)
    gj = jnp.zeros((64, 8), jnp.float32)
    gj = gj.at[0:32, 0:4].set(W1[4:8].T)
    gj = gj.at[32:64, 4:8].set(W1[4:8].T)
    return ws, wd, bw, w2, gi, gj


def kernel(x, edge_index, Wq1, bq1, Wq2, bq2, Wp1, bp1, Wp2, bp2, t_final):
    q = x[..., 0:4]
    p = x[..., 4:8]
    xi = x[..., 8:16]

    pad = jnp.full((E_PAD - E,), N, jnp.int32)
    i0p = jnp.concatenate([edge_index[0], pad]).reshape(IDX_ROWS, 128)
    i1p = jnp.concatenate([edge_index[1], pad]).reshape(IDX_ROWS, 128)
    zrows = jnp.zeros((ROWS_PER_TILE, 8), jnp.float32)

    mats_q = _make_mats(Wq1, bq1, Wq2, 0)
    mats_p = _make_mats(Wp1, bp1, Wp2, 4)

    body = jnp.concatenate([q[0], p[0], xi[0], q[1], p[1], xi[1]], axis=1)
    table = jnp.concatenate([body, jnp.zeros((N_PAD - N, 32), jnp.float32)], axis=0)

    def grad_parts(table, mats):
        fs, fd = _gather(table, i0p, i1p)
        gs, gd = _mlp(fs, fd, *mats)
        return _scatter(gs, gd, i0p, i1p, zrows)

    for _ in range(2):
        table = _update(table, grad_parts(table, mats_q), (4, 20), -1.0)
        table = _update(table, grad_parts(table, mats_p), (0, 16), 1.0)

    out = jnp.stack([table[:N, 0:16], table[:N, 16:32]])
    return out + jnp.asarray(t_final * 0, dtype=out.dtype)


# speedup vs baseline: 149.6581x; 1.0946x over previous
import functools

import jax
import jax.numpy as jnp
from jax import lax
from jax.experimental import pallas as pl
from jax.experimental.pallas import tpu as pltpu
from jax.experimental.pallas import tpu_sc as plsc

N = 50000
E = 800000
DT = 1.0
N_PAD = 50176
E_PAD = 819200
IDX_ROWS = E_PAD // 128
PER_W = E_PAD // 32
CHUNKS = PER_W // 1024
IROWS_W = PER_W // 128
ROWS_PER_TILE = N_PAD // 16


def _sc_mesh():
    return plsc.VectorSubcoreMesh(
        core_axis_name="c", subcore_axis_name="s", num_cores=2, num_subcores=16
    )


def _gather(table_a, table_b, i0p, i1p):
    @functools.partial(
        pl.kernel,
        out_type=[jax.ShapeDtypeStruct((E_PAD, 32), jnp.float32)] * 2,
        mesh=_sc_mesh(),
        compiler_params=pltpu.CompilerParams(use_tc_tiling_on_sc=False),
        scratch_types=[
            pltpu.VMEM((IROWS_W, 128), jnp.int32),
            pltpu.VMEM((IROWS_W, 128), jnp.int32),
            pltpu.VMEM((1024, 32), jnp.float32),
            pltpu.VMEM((1024, 32), jnp.float32),
            pltpu.SemaphoreType.DMA,
            pltpu.SemaphoreType.DMA,
        ],
    )
    def k(ta_h, tb_h, i0_h, i1_h, fs_h, fd_h, ia_v, ib_v, ra_v, rb_v, sa, sb):
        cid = lax.axis_index("c")
        wid = lax.axis_index("s") * 2 + cid
        pltpu.sync_copy(i0_h.at[pl.ds(wid * IROWS_W, IROWS_W)], ia_v)
        pltpu.sync_copy(i1_h.at[pl.ds(wid * IROWS_W, IROWS_W)], ib_v)

        def run(table_h):
            @pl.loop(0, CHUNKS)
            def _chunk(kk):
                cb = wid * PER_W + kk * 1024
                cps = [
                    pltpu.async_copy(table_h.at[ia_v.at[kk * 8 + j]],
                                     ra_v.at[pl.ds(j * 128, 128)], sa)
                    for j in range(8)
                ] + [
                    pltpu.async_copy(table_h.at[ib_v.at[kk * 8 + j]],
                                     rb_v.at[pl.ds(j * 128, 128)], sb)
                    for j in range(8)
                ]
                for c in cps:
                    c.wait()
                pltpu.sync_copy(ra_v, fs_h.at[pl.ds(cb, 1024)])
                pltpu.sync_copy(rb_v, fd_h.at[pl.ds(cb, 1024)])

        @pl.when(cid == 0)
        def _():
            run(ta_h)

        @pl.when(cid == 1)
        def _():
            run(tb_h)

    return k(table_a, table_b, i0p, i1p)


def _scatter(gs, gd, i0p, i1p, zrows):
    @functools.partial(
        pl.kernel,
        out_type=jax.ShapeDtypeStruct((2, N_PAD, 8), jnp.float32),
        mesh=_sc_mesh(),
        compiler_params=pltpu.CompilerParams(use_tc_tiling_on_sc=False),
        scratch_types=[
            pltpu.VMEM_SHARED((N_PAD, 8), jnp.float32),
            pltpu.VMEM((IROWS_W, 128), jnp.int32),
            pltpu.VMEM((IROWS_W, 128), jnp.int32),
            pltpu.VMEM((1024, 8), jnp.float32),
            pltpu.VMEM((1024, 8), jnp.float32),
            pltpu.VMEM((1024, 8), jnp.float32),
            pltpu.VMEM((1024, 8), jnp.float32),
            pltpu.SemaphoreType.DMA,
            pltpu.SemaphoreType.DMA,
        ],
    )
    def k(gs_h, gd_h, i0_h, i1_h, z_h, out_h,
          acc_sh, ia_v, ib_v, sa0, sb0, sa1, sb1, s_ld, s_add):
        cid = lax.axis_index("c")
        sid = lax.axis_index("s")
        wid = sid * 2 + cid
        pltpu.sync_copy(z_h, acc_sh.at[pl.ds(sid * ROWS_PER_TILE, ROWS_PER_TILE)])
        pltpu.sync_copy(i0_h.at[pl.ds(wid * IROWS_W, IROWS_W)], ia_v)
        pltpu.sync_copy(i1_h.at[pl.ds(wid * IROWS_W, IROWS_W)], ib_v)
        plsc.subcore_barrier()

        def load(kk, sa, sb):
            cb = wid * PER_W + kk * 1024
            return [
                pltpu.async_copy(gs_h.at[pl.ds(cb, 1024)], sa, s_ld),
                pltpu.async_copy(gd_h.at[pl.ds(cb, 1024)], sb, s_ld),
            ]

        def adds(kk, sa, sb):
            cps = [
                pltpu.async_copy(sa.at[pl.ds(j * 128, 128)], acc_sh.at[ia_v.at[kk * 8 + j]],
                                 s_add, add=True)
                for j in range(8)
            ] + [
                pltpu.async_copy(sb.at[pl.ds(j * 128, 128)], acc_sh.at[ib_v.at[kk * 8 + j]],
                                 s_add, add=True)
                for j in range(8)
            ]
            for c in cps:
                c.wait()

        for c in load(0, sa0, sb0):
            c.wait()

        @pl.loop(0, (CHUNKS - 1) // 2)
        def _pair(t):
            ka = 2 * t + 1
            l1 = load(ka, sa1, sb1)
            adds(2 * t, sa0, sb0)
            for c in l1:
                c.wait()
            l0 = load(ka + 1, sa0, sb0)
            adds(ka, sa1, sb1)
            for c in l0:
                c.wait()

        adds(CHUNKS - 1, sa0, sb0)

        plsc.subcore_barrier()
        pltpu.sync_copy(
            acc_sh.at[pl.ds(sid * ROWS_PER_TILE, ROWS_PER_TILE)],
            out_h.at[cid, pl.ds(sid * ROWS_PER_TILE, ROWS_PER_TILE)],
        )

    return k(gs, gd, i0p, i1p, zrows)


def _mlp_body(fs_r, fd_r, ws_r, wd_r, bw_r, w2_r, gi_r, gj_r, gs_o, gd_o):
    z = jnp.dot(fs_r[...], ws_r[...], preferred_element_type=jnp.float32)
    z = z + jnp.dot(fd_r[...], wd_r[...], preferred_element_type=jnp.float32)
    z = z + bw_r[...]
    h = jnp.tanh(z)
    g = (1.0 - h * h) * w2_r[...]
    gs_o[...] = jnp.dot(g, gi_r[...], preferred_element_type=jnp.float32)
    gd_o[...] = jnp.dot(g, gj_r[...], preferred_element_type=jnp.float32)


def _mlp(fs, fd, ws, wd, bw, w2, gi, gj):
    r = 2048
    rows = E_PAD // 4
    fs4 = fs.reshape(rows, 128)
    fd4 = fd.reshape(rows, 128)
    eye4 = jnp.eye(4, dtype=jnp.float32)
    ws4 = jnp.kron(eye4, ws)
    wd4 = jnp.kron(eye4, wd)
    bw4 = jnp.tile(bw, (1, 4))
    w24 = jnp.tile(w2, (1, 4))
    gi4 = jnp.kron(eye4, gi)
    gj4 = jnp.kron(eye4, gj)
    full = lambda shape: pl.BlockSpec(shape, lambda i: (0, 0))
    gs4, gd4 = pl.pallas_call(
        _mlp_body,
        grid=(rows // r,),
        in_specs=[
            pl.BlockSpec((r, 128), lambda i: (i, 0)),
            pl.BlockSpec((r, 128), lambda i: (i, 0)),
            full((128, 256)),
            full((128, 256)),
            full((1, 256)),
            full((1, 256)),
            full((256, 32)),
            full((256, 32)),
        ],
        out_specs=[
            pl.BlockSpec((r, 32), lambda i: (i, 0)),
            pl.BlockSpec((r, 32), lambda i: (i, 0)),
        ],
        out_shape=[jax.ShapeDtypeStruct((rows, 32), jnp.float32)] * 2,
    )(fs4, fd4, ws4, wd4, bw4, w24, gi4, gj4)
    return gs4.reshape(E_PAD, 8), gd4.reshape(E_PAD, 8)


def _update(table, parts, cols, sign):
    r = 3136
    c0, c1 = cols

    def body(t_r, pa_r, o_r, o2_r):
        acc = pa_r[0] + pa_r[1]
        t = t_r[...]
        d0 = sign * DT * acc[:, 0:4]
        d1 = sign * DT * acc[:, 4:8]
        pieces = []
        if c0 > 0:
            pieces.append(t[:, 0:c0])
        pieces.append(t[:, c0:c0 + 4] + d0)
        pieces.append(t[:, c0 + 4:c1])
        pieces.append(t[:, c1:c1 + 4] + d1)
        if c1 + 4 < 32:
            pieces.append(t[:, c1 + 4:32])
        new = jnp.concatenate(pieces, axis=1)
        o_r[...] = new
        o2_r[...] = new

    return pl.pallas_call(
        body,
        grid=(N_PAD // r,),
        in_specs=[
            pl.BlockSpec((r, 32), lambda i: (i, 0)),
            pl.BlockSpec((2, r, 8), lambda i: (0, i, 0)),
        ],
        out_specs=[
            pl.BlockSpec((r, 32), lambda i: (i, 0)),
            pl.BlockSpec((r, 32), lambda i: (i, 0)),
        ],
        out_shape=[jax.ShapeDtypeStruct((N_PAD, 32), jnp.float32)] * 2,
    )(table, parts)


def _make_mats(W1, b1, W2, vo):
    ws = jnp.zeros((32, 64), jnp.float32)
    wd = jnp.zeros((32, 64), jnp.float32)
    for bo, co in ((0, 0), (16, 32)):
        ws = ws.at[bo + vo:bo + vo + 4, co:co + 32].set(W1[0:4])
        ws = ws.at[bo + 8:bo + 16, co:co + 32].set(W1[8:16])
        wd = wd.at[bo + vo:bo + vo + 4, co:co + 32].set(W1[4:8])
        wd = wd.at[bo + 8:bo + 16, co:co + 32].set(W1[16:24])
    bw = jnp.concatenate([b1, b1]).reshape(1, 64)
    w2 = jnp.concatenate([W2[:, 0], W2[:, 0]]).reshape(1, 64)
    gi = jnp.zeros((64, 8), jnp.float32)
    gi = gi.at[0:32, 0:4].set(W1[0:4].T)
    gi = gi.at[32:64, 4:8].set(W1[0:4].T)
    gj = jnp.zeros((64, 8), jnp.float32)
    gj = gj.at[0:32, 0:4].set(W1[4:8].T)
    gj = gj.at[32:64, 4:8].set(W1[4:8].T)
    return ws, wd, bw, w2, gi, gj


def kernel(x, edge_index, Wq1, bq1, Wq2, bq2, Wp1, bp1, Wp2, bp2, t_final):
    q = x[..., 0:4]
    p = x[..., 4:8]
    xi = x[..., 8:16]

    pad = jnp.full((E_PAD - E,), N, jnp.int32)
    i0p = jnp.concatenate([edge_index[0], pad]).reshape(IDX_ROWS, 128)
    i1p = jnp.concatenate([edge_index[1], pad]).reshape(IDX_ROWS, 128)
    zrows = jnp.zeros((ROWS_PER_TILE, 8), jnp.float32)

    mats_q = _make_mats(Wq1, bq1, Wq2, 0)
    mats_p = _make_mats(Wp1, bp1, Wp2, 4)

    body = jnp.concatenate([q[0], p[0], xi[0], q[1], p[1], xi[1]], axis=1)
    table = jnp.concatenate([body, jnp.zeros((N_PAD - N, 32), jnp.float32)], axis=0)
    table_b = table + 0.0

    def grad_parts(table, table_b, mats):
        fs, fd = _gather(table, table_b, i0p, i1p)
        gs, gd = _mlp(fs, fd, *mats)
        return _scatter(gs, gd, i0p, i1p, zrows)

    for _ in range(2):
        table, table_b = _update(table, grad_parts(table, table_b, mats_q), (4, 20), -1.0)
        table, table_b = _update(table, grad_parts(table, table_b, mats_p), (0, 16), 1.0)

    out = jnp.stack([table[:N, 0:16], table[:N, 16:32]])
    return out + jnp.asarray(t_final * 0, dtype=out.dtype)


# speedup vs baseline: 150.4184x; 1.0051x over previous
import functools

import jax
import jax.numpy as jnp
from jax import lax
from jax.experimental import pallas as pl
from jax.experimental.pallas import tpu as pltpu
from jax.experimental.pallas import tpu_sc as plsc

N = 50000
E = 800000
DT = 1.0
N_PAD = 50176
E_PAD = 819200
IDX_ROWS = E_PAD // 128
PER_W = E_PAD // 32
CHUNKS = PER_W // 1024
IROWS_W = PER_W // 128
ROWS_PER_TILE = N_PAD // 16


def _sc_mesh():
    return plsc.VectorSubcoreMesh(
        core_axis_name="c", subcore_axis_name="s", num_cores=2, num_subcores=16
    )


GC = 512
GC_ROWS = GC // 128
CH0 = 70
CH1 = 30
BASE1 = 16 * CH0 * GC
assert 16 * (CH0 + CH1) * GC == E_PAD


def _gather(table_a, table_b, i0p, i1p):
    @functools.partial(
        pl.kernel,
        out_type=[jax.ShapeDtypeStruct((E_PAD, 32), jnp.float32)] * 2,
        mesh=_sc_mesh(),
        compiler_params=pltpu.CompilerParams(use_tc_tiling_on_sc=False),
        scratch_types=[
            pltpu.VMEM((2, GC_ROWS, 128), jnp.int32),
            pltpu.VMEM((2, GC_ROWS, 128), jnp.int32),
            pltpu.VMEM((GC, 32), jnp.float32),
            pltpu.VMEM((GC, 32), jnp.float32),
            pltpu.VMEM((GC, 32), jnp.float32),
            pltpu.VMEM((GC, 32), jnp.float32),
            pltpu.SemaphoreType.DMA,
            pltpu.SemaphoreType.DMA,
        ],
    )
    def k(ta_h, tb_h, i0_h, i1_h, fs_h, fd_h, ia_v, ib_v, ra0, rb0, ra1, rb1, sa, sb):
        cid = lax.axis_index("c")
        sid = lax.axis_index("s")
        rbufs = ((ra0, rb0), (ra1, rb1))

        def run(table_h, base_edge, base_row, nch):
            def sidx(kk, b):
                pltpu.sync_copy(i0_h.at[pl.ds(base_row + kk * GC_ROWS, GC_ROWS)], ia_v.at[b])
                pltpu.sync_copy(i1_h.at[pl.ds(base_row + kk * GC_ROWS, GC_ROWS)], ib_v.at[b])

            def fire(kk, b):
                ra, rb = rbufs[b]
                for j in range(GC_ROWS):
                    pltpu.async_copy(table_h.at[ia_v.at[b, j]], ra.at[pl.ds(j * 128, 128)], sa)
                for j in range(GC_ROWS):
                    pltpu.async_copy(table_h.at[ib_v.at[b, j]], rb.at[pl.ds(j * 128, 128)], sb)

            def drain(b):
                ra, rb = rbufs[b]
                for j in range(GC_ROWS):
                    pltpu.make_async_copy(table_h.at[ia_v.at[b, j]],
                                          ra.at[pl.ds(j * 128, 128)], sa).wait()
                for j in range(GC_ROWS):
                    pltpu.make_async_copy(table_h.at[ib_v.at[b, j]],
                                          rb.at[pl.ds(j * 128, 128)], sb).wait()

            def copyout(kk, b):
                ra, rb = rbufs[b]
                cb = base_edge + kk * GC
                pltpu.sync_copy(ra, fs_h.at[pl.ds(cb, GC)])
                pltpu.sync_copy(rb, fd_h.at[pl.ds(cb, GC)])

            sidx(0, 0)
            fire(0, 0)
            sidx(1, 1)

            @pl.loop(0, (nch - 2) // 2)
            def _pair(t):
                ka = 2 * t + 1
                drain(0)
                fire(ka, 1)
                copyout(ka - 1, 0)
                sidx(ka + 1, 0)
                drain(1)
                fire(ka + 1, 0)
                copyout(ka, 1)
                sidx(ka + 2, 1)

            fire(nch - 1, 1)
            drain(0)
            copyout(nch - 2, 0)
            drain(1)
            copyout(nch - 1, 1)

        @pl.when(cid == 0)
        def _():
            run(ta_h, sid * (CH0 * GC), sid * (CH0 * GC_ROWS), CH0)

        @pl.when(cid == 1)
        def _():
            run(tb_h, BASE1 + sid * (CH1 * GC), (BASE1 // 128) + sid * (CH1 * GC_ROWS), CH1)

    return k(table_a, table_b, i0p, i1p)


def _scatter(gs, gd, i0p, i1p, zrows):
    @functools.partial(
        pl.kernel,
        out_type=jax.ShapeDtypeStruct((2, N_PAD, 8), jnp.float32),
        mesh=_sc_mesh(),
        compiler_params=pltpu.CompilerParams(use_tc_tiling_on_sc=False),
        scratch_types=[
            pltpu.VMEM_SHARED((N_PAD, 8), jnp.float32),
            pltpu.VMEM((IROWS_W, 128), jnp.int32),
            pltpu.VMEM((IROWS_W, 128), jnp.int32),
            pltpu.VMEM((1024, 8), jnp.float32),
            pltpu.VMEM((1024, 8), jnp.float32),
            pltpu.VMEM((1024, 8), jnp.float32),
            pltpu.VMEM((1024, 8), jnp.float32),
            pltpu.SemaphoreType.DMA,
            pltpu.SemaphoreType.DMA,
        ],
    )
    def k(gs_h, gd_h, i0_h, i1_h, z_h, out_h,
          acc_sh, ia_v, ib_v, sa0, sb0, sa1, sb1, s_ld, s_add):
        cid = lax.axis_index("c")
        sid = lax.axis_index("s")
        wid = sid * 2 + cid
        pltpu.sync_copy(z_h, acc_sh.at[pl.ds(sid * ROWS_PER_TILE, ROWS_PER_TILE)])
        pltpu.sync_copy(i0_h.at[pl.ds(wid * IROWS_W, IROWS_W)], ia_v)
        pltpu.sync_copy(i1_h.at[pl.ds(wid * IROWS_W, IROWS_W)], ib_v)
        plsc.subcore_barrier()

        def load(kk, sa, sb):
            cb = wid * PER_W + kk * 1024
            return [
                pltpu.async_copy(gs_h.at[pl.ds(cb, 1024)], sa, s_ld),
                pltpu.async_copy(gd_h.at[pl.ds(cb, 1024)], sb, s_ld),
            ]

        def adds(kk, sa, sb):
            cps = [
                pltpu.async_copy(sa.at[pl.ds(j * 128, 128)], acc_sh.at[ia_v.at[kk * 8 + j]],
                                 s_add, add=True)
                for j in range(8)
            ] + [
                pltpu.async_copy(sb.at[pl.ds(j * 128, 128)], acc_sh.at[ib_v.at[kk * 8 + j]],
                                 s_add, add=True)
                for j in range(8)
            ]
            for c in cps:
                c.wait()

        for c in load(0, sa0, sb0):
            c.wait()

        @pl.loop(0, (CHUNKS - 1) // 2)
        def _pair(t):
            ka = 2 * t + 1
            l1 = load(ka, sa1, sb1)
            adds(2 * t, sa0, sb0)
            for c in l1:
                c.wait()
            l0 = load(ka + 1, sa0, sb0)
            adds(ka, sa1, sb1)
            for c in l0:
                c.wait()

        adds(CHUNKS - 1, sa0, sb0)

        plsc.subcore_barrier()
        pltpu.sync_copy(
            acc_sh.at[pl.ds(sid * ROWS_PER_TILE, ROWS_PER_TILE)],
            out_h.at[cid, pl.ds(sid * ROWS_PER_TILE, ROWS_PER_TILE)],
        )

    return k(gs, gd, i0p, i1p, zrows)


def _mlp_body(fs_r, fd_r, ws_r, wd_r, bw_r, w2_r, gi_r, gj_r, gs_o, gd_o):
    z = jnp.dot(fs_r[...], ws_r[...], preferred_element_type=jnp.float32)
    z = z + jnp.dot(fd_r[...], wd_r[...], preferred_element_type=jnp.float32)
    z = z + bw_r[...]
    h = jnp.tanh(z)
    g = (1.0 - h * h) * w2_r[...]
    gs_o[...] = jnp.dot(g, gi_r[...], preferred_element_type=jnp.float32)
    gd_o[...] = jnp.dot(g, gj_r[...], preferred_element_type=jnp.float32)


def _mlp(fs, fd, ws, wd, bw, w2, gi, gj):
    r = 2048
    rows = E_PAD // 4
    fs4 = fs.reshape(rows, 128)
    fd4 = fd.reshape(rows, 128)
    eye4 = jnp.eye(4, dtype=jnp.float32)
    ws4 = jnp.kron(eye4, ws)
    wd4 = jnp.kron(eye4, wd)
    bw4 = jnp.tile(bw, (1, 4))
    w24 = jnp.tile(w2, (1, 4))
    gi4 = jnp.kron(eye4, gi)
    gj4 = jnp.kron(eye4, gj)
    full = lambda shape: pl.BlockSpec(shape, lambda i: (0, 0))
    gs4, gd4 = pl.pallas_call(
        _mlp_body,
        grid=(rows // r,),
        in_specs=[
            pl.BlockSpec((r, 128), lambda i: (i, 0)),
            pl.BlockSpec((r, 128), lambda i: (i, 0)),
            full((128, 256)),
            full((128, 256)),
            full((1, 256)),
            full((1, 256)),
            full((256, 32)),
            full((256, 32)),
        ],
        out_specs=[
            pl.BlockSpec((r, 32), lambda i: (i, 0)),
            pl.BlockSpec((r, 32), lambda i: (i, 0)),
        ],
        out_shape=[jax.ShapeDtypeStruct((rows, 32), jnp.float32)] * 2,
    )(fs4, fd4, ws4, wd4, bw4, w24, gi4, gj4)
    return gs4.reshape(E_PAD, 8), gd4.reshape(E_PAD, 8)


def _update(table, parts, cols, sign):
    r = 3136
    c0, c1 = cols

    def body(t_r, pa_r, o_r, o2_r):
        acc = pa_r[0] + pa_r[1]
        t = t_r[...]
        d0 = sign * DT * acc[:, 0:4]
        d1 = sign * DT * acc[:, 4:8]
        pieces = []
        if c0 > 0:
            pieces.append(t[:, 0:c0])
        pieces.append(t[:, c0:c0 + 4] + d0)
        pieces.append(t[:, c0 + 4:c1])
        pieces.append(t[:, c1:c1 + 4] + d1)
        if c1 + 4 < 32:
            pieces.append(t[:, c1 + 4:32])
        new = jnp.concatenate(pieces, axis=1)
        o_r[...] = new
        o2_r[...] = new

    return pl.pallas_call(
        body,
        grid=(N_PAD // r,),
        in_specs=[
            pl.BlockSpec((r, 32), lambda i: (i, 0)),
            pl.BlockSpec((2, r, 8), lambda i: (0, i, 0)),
        ],
        out_specs=[
            pl.BlockSpec((r, 32), lambda i: (i, 0)),
            pl.BlockSpec((r, 32), lambda i: (i, 0)),
        ],
        out_shape=[jax.ShapeDtypeStruct((N_PAD, 32), jnp.float32)] * 2,
    )(table, parts)


def _make_mats(W1, b1, W2, vo):
    ws = jnp.zeros((32, 64), jnp.float32)
    wd = jnp.zeros((32, 64), jnp.float32)
    for bo, co in ((0, 0), (16, 32)):
        ws = ws.at[bo + vo:bo + vo + 4, co:co + 32].set(W1[0:4])
        ws = ws.at[bo + 8:bo + 16, co:co + 32].set(W1[8:16])
        wd = wd.at[bo + vo:bo + vo + 4, co:co + 32].set(W1[4:8])
        wd = wd.at[bo + 8:bo + 16, co:co + 32].set(W1[16:24])
    bw = jnp.concatenate([b1, b1]).reshape(1, 64)
    w2 = jnp.concatenate([W2[:, 0], W2[:, 0]]).reshape(1, 64)
    gi = jnp.zeros((64, 8), jnp.float32)
    gi = gi.at[0:32, 0:4].set(W1[0:4].T)
    gi = gi.at[32:64, 4:8].set(W1[0:4].T)
    gj = jnp.zeros((64, 8), jnp.float32)
    gj = gj.at[0:32, 0:4].set(W1[4:8].T)
    gj = gj.at[32:64, 4:8].set(W1[4:8].T)
    return ws, wd, bw, w2, gi, gj


def kernel(x, edge_index, Wq1, bq1, Wq2, bq2, Wp1, bp1, Wp2, bp2, t_final):
    q = x[..., 0:4]
    p = x[..., 4:8]
    xi = x[..., 8:16]

    pad = jnp.full((E_PAD - E,), N, jnp.int32)
    i0p = jnp.concatenate([edge_index[0], pad]).reshape(IDX_ROWS, 128)
    i1p = jnp.concatenate([edge_index[1], pad]).reshape(IDX_ROWS, 128)
    zrows = jnp.zeros((ROWS_PER_TILE, 8), jnp.float32)

    mats_q = _make_mats(Wq1, bq1, Wq2, 0)
    mats_p = _make_mats(Wp1, bp1, Wp2, 4)

    body = jnp.concatenate([q[0], p[0], xi[0], q[1], p[1], xi[1]], axis=1)
    table = jnp.concatenate([body, jnp.zeros((N_PAD - N, 32), jnp.float32)], axis=0)
    table_b = table + 0.0

    def grad_parts(table, table_b, mats):
        fs, fd = _gather(table, table_b, i0p, i1p)
        gs, gd = _mlp(fs, fd, *mats)
        return _scatter(gs, gd, i0p, i1p, zrows)

    for _ in range(2):
        table, table_b = _update(table, grad_parts(table, table_b, mats_q), (4, 20), -1.0)
        table, table_b = _update(table, grad_parts(table, table_b, mats_p), (0, 16), 1.0)

    out = jnp.stack([table[:N, 0:16], table[:N, 16:32]])
    return out + jnp.asarray(t_final * 0, dtype=out.dtype)
